# baseline (speedup 1.0000x reference)
import functools

import jax
import jax.numpy as jnp
from jax.experimental import pallas as pl
from jax.experimental.pallas import tpu as pltpu

B, S, D = 1, 2048, 1024
FF = 3584
E = 8
TOP_K = 2

FFC = 512
NF = FF // FFC


def _router_body(h_ref, wg_ref, ew_ref):
    h = h_ref[...]
    wg = wg_ref[...]
    logits = jnp.dot(h, wg, preferred_element_type=jnp.float32)
    m = jnp.max(logits, axis=1, keepdims=True)
    ex = jnp.exp(logits - m)
    p = ex / jnp.sum(ex, axis=1, keepdims=True)
    idx = jax.lax.broadcasted_iota(jnp.int32, (S, E), 1)
    v0 = jnp.max(p, axis=1, keepdims=True)
    e0 = jnp.min(jnp.where(p == v0, idx, E), axis=1, keepdims=True)
    p1 = jnp.where(idx == e0, -jnp.inf, p)
    v1 = jnp.max(p1, axis=1, keepdims=True)
    e1 = jnp.min(jnp.where(p1 == v1, idx, E), axis=1, keepdims=True)
    s = v0 + v1
    ew_ref[...] = jnp.where(idx == e0, v0 / s, 0.0) + jnp.where(idx == e1, v1 / s, 0.0)


def _moe_body(ew_ref, h_ref, wup_ref, wgate_ref, wdown_ref, out_ref):
    e = pl.program_id(0)
    f = pl.program_id(1)

    @pl.when((e == 0) & (f == 0))
    def _():
        out_ref[...] = jnp.zeros_like(out_ref)

    x = h_ref[...]
    up = jnp.dot(x, wup_ref[0], preferred_element_type=jnp.float32)
    gate = jnp.dot(x, wgate_ref[0], preferred_element_type=jnp.float32)
    z = up * jax.nn.sigmoid(up) * gate

    idx = jax.lax.broadcasted_iota(jnp.int32, (S, E), 1)
    w_col = jnp.sum(jnp.where(idx == e, ew_ref[...], 0.0), axis=1, keepdims=True)
    zw = (z * w_col).astype(jnp.bfloat16)
    out_ref[...] += jnp.dot(zw, wdown_ref[0], preferred_element_type=jnp.float32)


@jax.jit
def _run(h2d, Wg, W_up, W_gate, W_down):
    ew = pl.pallas_call(
        _router_body,
        out_shape=jax.ShapeDtypeStruct((S, E), jnp.float32),
    )(h2d, Wg)

    h_bf = h2d.astype(jnp.bfloat16)
    out = pl.pallas_call(
        _moe_body,
        grid=(E, NF),
        in_specs=[
            pl.BlockSpec((S, E), lambda e, f: (0, 0)),
            pl.BlockSpec((S, D), lambda e, f: (0, 0)),
            pl.BlockSpec((1, D, FFC), lambda e, f: (e, 0, f)),
            pl.BlockSpec((1, D, FFC), lambda e, f: (e, 0, f)),
            pl.BlockSpec((1, FFC, D), lambda e, f: (e, f, 0)),
        ],
        out_specs=pl.BlockSpec((S, D), lambda e, f: (0, 0)),
        out_shape=jax.ShapeDtypeStruct((S, D), jnp.float32),
    )(ew, h_bf, W_up, W_gate, W_down)
    return out


def kernel(hidden_states, Wg, W_up, W_gate, W_down):
    h2d = hidden_states.reshape(-1, D)
    out = _run(
        h2d,
        Wg,
        W_up.astype(jnp.bfloat16),
        W_gate.astype(jnp.bfloat16),
        W_down.astype(jnp.bfloat16),
    )
    return out.reshape(hidden_states.shape)


# speedup vs baseline: 1.4210x; 1.4210x over previous
import functools

import jax
import jax.numpy as jnp
from jax import lax
from jax.experimental import pallas as pl
from jax.experimental.pallas import tpu as pltpu
from jax.experimental.pallas import tpu_sc as plsc

B, S, D = 1, 2048, 1024
FF = 3584
E = 8
TOP_K = 2

T = 256
MAXG = (TOP_K * S) // T + E
PAD_ROWS = MAXG * T
FFC = 1792
NF = FF // FFC

NW = 32
TPW = S // NW
CH = 32
LANES = 16
WREP = 128


def _router_meta_body(h_ref, wg_ref, dst_ref, wrep_ref, counts_ref):
    h = h_ref[...]
    logits = jnp.dot(h, wg_ref[...], preferred_element_type=jnp.float32)
    m = jnp.max(logits, axis=1, keepdims=True)
    ex = jnp.exp(logits - m)
    p = ex / jnp.sum(ex, axis=1, keepdims=True)
    idx = lax.broadcasted_iota(jnp.int32, (S, E), 1)
    v0 = jnp.max(p, axis=1, keepdims=True)
    e0 = jnp.min(jnp.where(p == v0, idx, E), axis=1, keepdims=True)
    p1 = jnp.where(idx == e0, -jnp.inf, p)
    v1 = jnp.max(p1, axis=1, keepdims=True)
    e1 = jnp.min(jnp.where(p1 == v1, idx, E), axis=1, keepdims=True)
    s = v0 + v1
    w_a = jnp.concatenate([v0 / s, v1 / s], axis=0)
    wrep_ref[...] = jnp.broadcast_to(w_a, (TOP_K * S, WREP))

    e_a = jnp.concatenate([e0, e1], axis=0)
    idx2 = lax.broadcasted_iota(jnp.int32, (TOP_K * S, E), 1)
    oh = (e_a == idx2).astype(jnp.float32)

    RB = 512
    ri = lax.broadcasted_iota(jnp.int32, (RB, RB), 0)
    ci = lax.broadcasted_iota(jnp.int32, (RB, RB), 1)
    ltri = (ci < ri).astype(jnp.float32)
    carry = jnp.zeros((1, E), jnp.float32)
    ranks = []
    for b in range(TOP_K * S // RB):
        ohb = oh[b * RB:(b + 1) * RB, :]
        cb = jnp.dot(ltri, ohb, preferred_element_type=jnp.float32) + carry
        ranks.append(jnp.sum(cb * ohb, axis=1, keepdims=True))
        carry = carry + jnp.sum(ohb, axis=0, keepdims=True)
    rank_a = jnp.concatenate(ranks, axis=0)

    counts = carry
    ntiles = jnp.floor((counts + (T - 1)) / T)
    ri8 = lax.broadcasted_iota(jnp.int32, (E, E), 0)
    ci8 = lax.broadcasted_iota(jnp.int32, (E, E), 1)
    utri = (ri8 < ci8).astype(jnp.float32)
    cum_excl = jnp.dot(ntiles, utri, preferred_element_type=jnp.float32)
    poff = cum_excl * T
    poff_a = jnp.sum(oh * poff, axis=1, keepdims=True)
    dst_ref[...] = (rank_a + poff_a).astype(jnp.int32)
    counts_ref[...] = counts.astype(jnp.int32)


def _gmm_body(texp_ref, tact_ref, xs_ref, ws_ref, wup_ref, wgate_ref, wdown_ref,
              out_ref):
    f = pl.program_id(1)

    @pl.when(f == 0)
    def _():
        out_ref[...] = jnp.zeros_like(out_ref)

    @pl.when(tact_ref[pl.program_id(0)] == 1)
    def _():
        x = xs_ref[...].astype(jnp.bfloat16)
        up = jnp.dot(x, wup_ref[0], preferred_element_type=jnp.float32)
        gate = jnp.dot(x, wgate_ref[0], preferred_element_type=jnp.float32)
        z = (up * jax.nn.sigmoid(up) * gate).astype(jnp.bfloat16)
        out_ref[...] += jnp.dot(z, wdown_ref[0], preferred_element_type=jnp.float32)

        @pl.when(f == NF - 1)
        def _():
            out_ref[...] *= ws_ref[...][:, 0:1]


@functools.cache
def _make_scatter_sc():
    mesh = plsc.VectorSubcoreMesh(core_axis_name="c", subcore_axis_name="s")

    @functools.partial(
        pl.kernel,
        out_type=[
            jax.ShapeDtypeStruct((PAD_ROWS, D), jnp.float32),
            jax.ShapeDtypeStruct((PAD_ROWS, WREP), jnp.float32),
        ],
        mesh=mesh,
        scratch_types=[
            pltpu.VMEM((TPW,), jnp.int32),
            pltpu.VMEM((TPW,), jnp.int32),
            pltpu.VMEM((TPW, D), jnp.float32),
            pltpu.VMEM((TPW, WREP), jnp.float32),
            pltpu.VMEM((TPW, WREP), jnp.float32),
            pltpu.SemaphoreType.DMA,
        ],
    )
    def _scatter_sc(h_hbm, dst_hbm, wrep_hbm, xs_hbm, ws_hbm,
                    idx0_v, idx1_v, rows_v, w0_v, w1_v, sem):
        wid = lax.axis_index("s") * 2 + lax.axis_index("c")
        base = wid * TPW
        pltpu.sync_copy(dst_hbm.at[pl.ds(base, TPW)], idx0_v)
        pltpu.sync_copy(dst_hbm.at[pl.ds(S + base, TPW)], idx1_v)
        pltpu.sync_copy(h_hbm.at[pl.ds(base, TPW)], rows_v)
        pltpu.sync_copy(wrep_hbm.at[pl.ds(base, TPW)], w0_v)
        pltpu.sync_copy(wrep_hbm.at[pl.ds(S + base, TPW)], w1_v)
        pltpu.async_copy(rows_v, xs_hbm.at[idx0_v], sem).wait()
        pltpu.async_copy(rows_v, xs_hbm.at[idx1_v], sem).wait()
        pltpu.async_copy(w0_v, ws_hbm.at[idx0_v], sem).wait()
        pltpu.async_copy(w1_v, ws_hbm.at[idx1_v], sem).wait()

    return _scatter_sc


@functools.cache
def _make_combine_sc():
    mesh = plsc.VectorSubcoreMesh(core_axis_name="c", subcore_axis_name="s")

    @functools.partial(
        pl.kernel,
        out_type=jax.ShapeDtypeStruct((S, D), jnp.float32),
        mesh=mesh,
        scratch_types=[
            pltpu.VMEM((CH,), jnp.int32),
            pltpu.VMEM((CH,), jnp.int32),
            pltpu.VMEM((CH, D), jnp.float32),
            pltpu.VMEM((CH, D), jnp.float32),
            pltpu.SemaphoreType.DMA,
        ],
    )
    def _combine_sc(y_hbm, dst_hbm, out_hbm, i0, i1, abuf, bbuf, sem):
        wid = lax.axis_index("s") * 2 + lax.axis_index("c")
        for ch in range(TPW // CH):
            base = wid * TPW + ch * CH
            pltpu.sync_copy(dst_hbm.at[pl.ds(base, CH)], i0)
            pltpu.sync_copy(dst_hbm.at[pl.ds(S + base, CH)], i1)
            pltpu.async_copy(y_hbm.at[i0], abuf, sem).wait()
            pltpu.async_copy(y_hbm.at[i1], bbuf, sem).wait()

            def row_body(r, _):
                def col_body(c, _):
                    av = abuf[r, pl.ds(c * LANES, LANES)]
                    bv = bbuf[r, pl.ds(c * LANES, LANES)]
                    abuf[r, pl.ds(c * LANES, LANES)] = av + bv
                    return 0

                lax.fori_loop(0, D // LANES, col_body, 0)
                return 0

            lax.fori_loop(0, CH, row_body, 0)
            pltpu.sync_copy(abuf, out_hbm.at[pl.ds(base, CH)])

    return _combine_sc


@jax.jit
def _run(h2d, Wg, W_up, W_gate, W_down):
    dst, wrep, counts2 = pl.pallas_call(
        _router_meta_body,
        out_shape=[
            jax.ShapeDtypeStruct((TOP_K * S, 1), jnp.int32),
            jax.ShapeDtypeStruct((TOP_K * S, WREP), jnp.float32),
            jax.ShapeDtypeStruct((1, E), jnp.int32),
        ],
    )(h2d, Wg)
    dst_flat = dst.reshape(TOP_K * S)

    counts = counts2[0]
    ntiles = (counts + T - 1) // T
    tcum = jnp.cumsum(ntiles)
    total = tcum[E - 1]
    gi = jnp.arange(MAXG, dtype=jnp.int32)
    texp_raw = jnp.sum(
        (gi[:, None] >= tcum[None, :]).astype(jnp.int32), axis=1
    ).astype(jnp.int32)
    emax = jnp.max(
        jnp.where(ntiles > 0, jnp.arange(E, dtype=jnp.int32), -1)
    ).astype(jnp.int32)
    texp = jnp.minimum(texp_raw, emax)
    tact = (gi < total).astype(jnp.int32)

    x_sorted, ws_sorted = _make_scatter_sc()(h2d, dst_flat, wrep)

    def _wf(g, f, texp_r, tact_r):
        return jnp.where(tact_r[g] == 1, f, NF - 1)

    y = pl.pallas_call(
        _gmm_body,
        grid_spec=pltpu.PrefetchScalarGridSpec(
            num_scalar_prefetch=2,
            grid=(MAXG, NF),
            in_specs=[
                pl.BlockSpec((T, D), lambda g, f, texp_r, tact_r: (g, 0)),
                pl.BlockSpec((T, WREP), lambda g, f, texp_r, tact_r: (g, 0)),
                pl.BlockSpec(
                    (1, D, FFC),
                    lambda g, f, texp_r, tact_r: (texp_r[g], 0, _wf(g, f, texp_r, tact_r)),
                ),
                pl.BlockSpec(
                    (1, D, FFC),
                    lambda g, f, texp_r, tact_r: (texp_r[g], 0, _wf(g, f, texp_r, tact_r)),
                ),
                pl.BlockSpec(
                    (1, FFC, D),
                    lambda g, f, texp_r, tact_r: (texp_r[g], _wf(g, f, texp_r, tact_r), 0),
                ),
            ],
            out_specs=pl.BlockSpec((T, D), lambda g, f, texp_r, tact_r: (g, 0)),
        ),
        out_shape=jax.ShapeDtypeStruct((PAD_ROWS, D), jnp.float32),
    )(
        texp,
        tact,
        x_sorted,
        ws_sorted,
        W_up.astype(jnp.bfloat16),
        W_gate.astype(jnp.bfloat16),
        W_down.astype(jnp.bfloat16),
    )

    return _make_combine_sc()(y, dst_flat)


def kernel(hidden_states, Wg, W_up, W_gate, W_down):
    h2d = hidden_states.reshape(-1, D)
    out = _run(h2d, Wg, W_up, W_gate, W_down)
    return out.reshape(hidden_states.shape)


# speedup vs baseline: 1.4596x; 1.0272x over previous
import functools

import jax
import jax.numpy as jnp
from jax import lax
from jax.experimental import pallas as pl
from jax.experimental.pallas import tpu as pltpu
from jax.experimental.pallas import tpu_sc as plsc

B, S, D = 1, 2048, 1024
FF = 3584
E = 8
TOP_K = 2

T = 128
MAXG = (TOP_K * S) // T + E
PAD_ROWS = MAXG * T

NW = 32
TPW = S // NW
CH = 32
LANES = 16
WREP = 128


def _router_meta_body(h_ref, wg_ref, dst_ref, wrep_ref, texp_ref, tact_ref):
    h = h_ref[...]
    logits = jnp.dot(h, wg_ref[...], preferred_element_type=jnp.float32)
    m = jnp.max(logits, axis=1, keepdims=True)
    ex = jnp.exp(logits - m)
    p = ex / jnp.sum(ex, axis=1, keepdims=True)
    idx = lax.broadcasted_iota(jnp.int32, (S, E), 1)
    v0 = jnp.max(p, axis=1, keepdims=True)
    e0 = jnp.min(jnp.where(p == v0, idx, E), axis=1, keepdims=True)
    p1 = jnp.where(idx == e0, -jnp.inf, p)
    v1 = jnp.max(p1, axis=1, keepdims=True)
    e1 = jnp.min(jnp.where(p1 == v1, idx, E), axis=1, keepdims=True)
    s = v0 + v1
    w_a = jnp.concatenate([v0 / s, v1 / s], axis=0)
    wrep_ref[...] = jnp.broadcast_to(w_a, (TOP_K * S, WREP))

    e_a = jnp.concatenate([e0, e1], axis=0)
    idx2 = lax.broadcasted_iota(jnp.int32, (TOP_K * S, E), 1)
    oh = (e_a == idx2).astype(jnp.float32)

    RB = 512
    ri = lax.broadcasted_iota(jnp.int32, (RB, RB), 0)
    ci = lax.broadcasted_iota(jnp.int32, (RB, RB), 1)
    ltri = (ci < ri).astype(jnp.float32)
    carry = jnp.zeros((1, E), jnp.float32)
    ranks = []
    for b in range(TOP_K * S // RB):
        ohb = oh[b * RB:(b + 1) * RB, :]
        cb = jnp.dot(ltri, ohb, preferred_element_type=jnp.float32) + carry
        ranks.append(jnp.sum(cb * ohb, axis=1, keepdims=True))
        carry = carry + jnp.sum(ohb, axis=0, keepdims=True)
    rank_a = jnp.concatenate(ranks, axis=0)

    counts = carry
    ntiles = jnp.floor((counts + (T - 1)) / T)
    ri8 = lax.broadcasted_iota(jnp.int32, (E, E), 0)
    ci8 = lax.broadcasted_iota(jnp.int32, (E, E), 1)
    utri = (ri8 < ci8).astype(jnp.float32)
    cum_excl = jnp.dot(ntiles, utri, preferred_element_type=jnp.float32)
    poff = cum_excl * T
    poff_a = jnp.sum(oh * poff, axis=1, keepdims=True)
    dst_ref[...] = (rank_a + poff_a).astype(jnp.int32)

    ones_col = jnp.ones((TOP_K * S, 1), jnp.float32)
    counts_col = lax.dot_general(
        oh, ones_col, (((0,), (0,)), ((), ())),
        preferred_element_type=jnp.float32)
    ntiles_col = jnp.floor((counts_col + (T - 1)) / T)
    ltri8_inc = (ri8 >= ci8).astype(jnp.float32)
    cum_inc_col = jnp.dot(ltri8_inc, ntiles_col,
                          preferred_element_type=jnp.float32)
    cum_i = cum_inc_col.astype(jnp.int32)
    gi = lax.broadcasted_iota(jnp.int32, (E, MAXG), 1)
    texp_raw = jnp.sum((gi >= cum_i).astype(jnp.int32),
                       axis=0, keepdims=True)
    e_col = lax.broadcasted_iota(jnp.int32, (E, 1), 0)
    emax = jnp.max(jnp.where(ntiles_col > 0.5, e_col, -1))
    texp_ref[...] = jnp.minimum(texp_raw, emax)
    total = jnp.max(cum_i)
    tact_ref[...] = (gi[0:1, :] < total).astype(jnp.int32)


def _gmm_body(texp_ref, tact_ref, xs_ref, ws_ref, wup_ref, wgate_ref, wdown_ref,
              out_ref):
    active = tact_ref[0, pl.program_id(0)] == 1

    @pl.when(jnp.logical_not(active))
    def _():
        out_ref[...] = jnp.zeros_like(out_ref)

    @pl.when(active)
    def _():
        x = xs_ref[...].astype(jnp.bfloat16)
        up = jnp.dot(x, wup_ref[0], preferred_element_type=jnp.float32)
        gate = jnp.dot(x, wgate_ref[0], preferred_element_type=jnp.float32)
        z = (up * jax.nn.sigmoid(up) * gate).astype(jnp.bfloat16)
        out_ref[...] = jnp.dot(
            z, wdown_ref[0], preferred_element_type=jnp.float32
        ) * ws_ref[...][:, 0:1]


@functools.cache
def _make_scatter_sc():
    mesh = plsc.VectorSubcoreMesh(core_axis_name="c", subcore_axis_name="s")

    @functools.partial(
        pl.kernel,
        out_type=[
            jax.ShapeDtypeStruct((PAD_ROWS, D), jnp.float32),
            jax.ShapeDtypeStruct((PAD_ROWS, WREP), jnp.float32),
        ],
        mesh=mesh,
        scratch_types=[
            pltpu.VMEM((TPW,), jnp.int32),
            pltpu.VMEM((TPW,), jnp.int32),
            pltpu.VMEM((TPW, D), jnp.float32),
            pltpu.VMEM((TPW, WREP), jnp.float32),
            pltpu.VMEM((TPW, WREP), jnp.float32),
            pltpu.SemaphoreType.DMA,
        ],
    )
    def _scatter_sc(h_hbm, dst_hbm, wrep_hbm, xs_hbm, ws_hbm,
                    idx0_v, idx1_v, rows_v, w0_v, w1_v, sem):
        wid = lax.axis_index("s") * 2 + lax.axis_index("c")
        base = wid * TPW
        pltpu.sync_copy(dst_hbm.at[pl.ds(base, TPW)], idx0_v)
        pltpu.sync_copy(dst_hbm.at[pl.ds(S + base, TPW)], idx1_v)
        pltpu.sync_copy(h_hbm.at[pl.ds(base, TPW)], rows_v)
        pltpu.sync_copy(wrep_hbm.at[pl.ds(base, TPW)], w0_v)
        pltpu.sync_copy(wrep_hbm.at[pl.ds(S + base, TPW)], w1_v)
        pltpu.async_copy(rows_v, xs_hbm.at[idx0_v], sem).wait()
        pltpu.async_copy(rows_v, xs_hbm.at[idx1_v], sem).wait()
        pltpu.async_copy(w0_v, ws_hbm.at[idx0_v], sem).wait()
        pltpu.async_copy(w1_v, ws_hbm.at[idx1_v], sem).wait()

    return _scatter_sc


@functools.cache
def _make_combine_sc():
    mesh = plsc.VectorSubcoreMesh(core_axis_name="c", subcore_axis_name="s")

    @functools.partial(
        pl.kernel,
        out_type=jax.ShapeDtypeStruct((S, D), jnp.float32),
        mesh=mesh,
        scratch_types=[
            pltpu.VMEM((CH,), jnp.int32),
            pltpu.VMEM((CH,), jnp.int32),
            pltpu.VMEM((CH, D), jnp.float32),
            pltpu.VMEM((CH, D), jnp.float32),
            pltpu.SemaphoreType.DMA,
        ],
    )
    def _combine_sc(y_hbm, dst_hbm, out_hbm, i0, i1, abuf, bbuf, sem):
        wid = lax.axis_index("s") * 2 + lax.axis_index("c")
        for ch in range(TPW // CH):
            base = wid * TPW + ch * CH
            pltpu.sync_copy(dst_hbm.at[pl.ds(base, CH)], i0)
            pltpu.sync_copy(dst_hbm.at[pl.ds(S + base, CH)], i1)
            pltpu.async_copy(y_hbm.at[i0], abuf, sem).wait()
            pltpu.async_copy(y_hbm.at[i1], bbuf, sem).wait()

            def row_body(r, _):
                def col_body(c, _):
                    av = abuf[r, pl.ds(c * LANES, LANES)]
                    bv = bbuf[r, pl.ds(c * LANES, LANES)]
                    abuf[r, pl.ds(c * LANES, LANES)] = av + bv
                    return 0

                lax.fori_loop(0, D // LANES, col_body, 0)
                return 0

            lax.fori_loop(0, CH, row_body, 0)
            pltpu.sync_copy(abuf, out_hbm.at[pl.ds(base, CH)])

    return _combine_sc


@jax.jit
def _run(h2d, Wg, W_up, W_gate, W_down):
    dst, wrep, texp, tact = pl.pallas_call(
        _router_meta_body,
        out_shape=[
            jax.ShapeDtypeStruct((TOP_K * S, 1), jnp.int32),
            jax.ShapeDtypeStruct((TOP_K * S, WREP), jnp.float32),
            jax.ShapeDtypeStruct((1, MAXG), jnp.int32),
            jax.ShapeDtypeStruct((1, MAXG), jnp.int32),
        ],
    )(h2d, Wg)
    dst_flat = dst.reshape(TOP_K * S)

    x_sorted, ws_sorted = _make_scatter_sc()(h2d, dst_flat, wrep)

    y = pl.pallas_call(
        _gmm_body,
        grid_spec=pltpu.PrefetchScalarGridSpec(
            num_scalar_prefetch=2,
            grid=(MAXG,),
            in_specs=[
                pl.BlockSpec((T, D), lambda g, texp_r, tact_r: (g, 0)),
                pl.BlockSpec((T, WREP), lambda g, texp_r, tact_r: (g, 0)),
                pl.BlockSpec(
                    (1, D, FF),
                    lambda g, texp_r, tact_r: (texp_r[0, g], 0, 0),
                ),
                pl.BlockSpec(
                    (1, D, FF),
                    lambda g, texp_r, tact_r: (texp_r[0, g], 0, 0),
                ),
                pl.BlockSpec(
                    (1, FF, D),
                    lambda g, texp_r, tact_r: (texp_r[0, g], 0, 0),
                ),
            ],
            out_specs=pl.BlockSpec((T, D), lambda g, texp_r, tact_r: (g, 0)),
        ),
        out_shape=jax.ShapeDtypeStruct((PAD_ROWS, D), jnp.float32),
    )(
        texp,
        tact,
        x_sorted,
        ws_sorted,
        W_up.astype(jnp.bfloat16),
        W_gate.astype(jnp.bfloat16),
        W_down.astype(jnp.bfloat16),
    )

    return _make_combine_sc()(y, dst_flat)


def kernel(hidden_states, Wg, W_up, W_gate, W_down):
    h2d = hidden_states.reshape(-1, D)
    out = _run(h2d, Wg, W_up, W_gate, W_down)
    return out.reshape(hidden_states.shape)
